# Initial kernel scaffold; baseline (speedup 1.0000x reference)
import jax
import jax.numpy as jnp
from jax.experimental import pallas as pl


def kernel(tx_x, card_x, addr_x, email_x, device_x, product_x, ei_to_card, ei_from_card, ei_to_addr, ei_from_addr, ei_to_email, ei_from_email, ei_to_device, ei_from_device, ei_to_product, ei_from_product, ei_tx_pattern, params):
    raise NotImplementedError("write your pallas kernel here")


# speedup vs baseline: 2.2311x; 2.2311x over previous
import functools
import jax
import jax.numpy as jnp
from jax import lax
from jax.experimental import pallas as pl
from jax.experimental.pallas import tpu as pltpu
from jax.experimental.pallas import tpu_sc as plsc

N_TX = 50000
N_ENT = 10000
H = 128
NCHUNK = 4
CW = 32
NS = 16
NC = 2

TX_PAD = 50176
ENT_PAD = 10240
ENT_ALL = 5 * ENT_PAD

TO_ACC = 51328
FR_STRIDE = 10248
FR_ACC = 51328
PAT_ACC = 50304

TO_DUMP = TO_ACC - 1
FR_DUMP = FR_ACC - 1
PAT_DUMP = PAT_ACC - 1

TO_E = 251904
FR_E = 251904
PAT_E = 100352

BN = 512


def _elu(x):
  return jnp.where(x > 0, x, jnp.exp(x) - 1.0)


def _inv_counts(cblk):
  c = cblk[:, 0:1]
  return 1.0 / jnp.maximum(c, 1.0)


def _chunk_writes(y, out_refs):
  for f in range(NCHUNK):
    out_refs[f][...] = y[:, f * CW:(f + 1) * CW]


def _concat_chunks(refs):
  return jnp.concatenate([refs[f][...] for f in range(NCHUNK)], axis=1)


def _tx_proj_body(x_ref, w1_ref, b1_ref, w2_ref, b2_ref, *out_refs):
  h = jnp.maximum(
      jnp.dot(x_ref[...], w1_ref[...], preferred_element_type=jnp.float32)
      + b1_ref[...], 0.0)
  y = _elu(
      jnp.dot(h, w2_ref[...], preferred_element_type=jnp.float32)
      + b2_ref[...])
  _chunk_writes(y, out_refs)


def _tx_proj(tx_pad, w1t, b1, w2t, b2):
  grid = TX_PAD // BN
  return pl.pallas_call(
      _tx_proj_body,
      grid=(grid,),
      in_specs=[
          pl.BlockSpec((BN, H), lambda i: (i, 0)),
          pl.BlockSpec((H, H), lambda i: (0, 0)),
          pl.BlockSpec((1, H), lambda i: (0, 0)),
          pl.BlockSpec((H, H), lambda i: (0, 0)),
          pl.BlockSpec((1, H), lambda i: (0, 0)),
      ],
      out_specs=[pl.BlockSpec((BN, CW), lambda i: (i, 0))] * NCHUNK,
      out_shape=[jax.ShapeDtypeStruct((TX_PAD, CW), jnp.float32)] * NCHUNK,
  )(tx_pad, w1t, b1, w2t, b2)


def _ent_proj_body(x_ref, we_ref, be_ref, *out_refs):
  y = jnp.maximum(
      jnp.dot(x_ref[...], we_ref[...], preferred_element_type=jnp.float32)
      + be_ref[...], 0.0)
  _chunk_writes(y, out_refs)


def _ent_proj(ent_all, wet, be):
  grid = ENT_ALL // BN
  return pl.pallas_call(
      _ent_proj_body,
      grid=(grid,),
      in_specs=[
          pl.BlockSpec((BN, 2), lambda i: (i, 0)),
          pl.BlockSpec((2, H), lambda i: (0, 0)),
          pl.BlockSpec((1, H), lambda i: (0, 0)),
      ],
      out_specs=[pl.BlockSpec((BN, CW), lambda i: (i, 0))] * NCHUNK,
      out_shape=[jax.ShapeDtypeStruct((ENT_ALL, CW), jnp.float32)] * NCHUNK,
  )(ent_all, wet, be)


def _ent_update_body(s0, s1, s2, s3, cto, x0, x1, x2, x3, wlt, bl, wrt,
                     *out_refs):
  agg = _concat_chunks((s0, s1, s2, s3)) * _inv_counts(cto[...])
  x = _concat_chunks((x0, x1, x2, x3))
  y = _elu(
      jnp.dot(agg, wlt[0], preferred_element_type=jnp.float32)
      + bl[0]
      + jnp.dot(x, wrt[0], preferred_element_type=jnp.float32))
  _chunk_writes(y, out_refs)


def _ent_update(s_to, c_to, x_ent, wlt_to, bl_to, wrt_to):
  grid = ENT_ALL // BN
  blocks_per_et = ENT_PAD // BN
  sspec = pl.BlockSpec((BN, CW), lambda i: (i, 0))
  return pl.pallas_call(
      _ent_update_body,
      grid=(grid,),
      in_specs=(
          [sspec] * NCHUNK
          + [pl.BlockSpec((BN, 16), lambda i: (i, 0))]
          + [sspec] * NCHUNK
          + [pl.BlockSpec((1, H, H), lambda i: (i // blocks_per_et, 0, 0)),
             pl.BlockSpec((1, 1, H), lambda i: (i // blocks_per_et, 0, 0)),
             pl.BlockSpec((1, H, H), lambda i: (i // blocks_per_et, 0, 0))]
      ),
      out_specs=[sspec] * NCHUNK,
      out_shape=[jax.ShapeDtypeStruct((ENT_ALL, CW), jnp.float32)] * NCHUNK,
  )(*s_to, c_to, *x_ent, wlt_to, bl_to, wrt_to)


def _tx_update_body(head, nlow, f0, f1, f2, f3, cfr, p0, p1, p2, p3, cpat,
                    x0, x1, x2, x3, wlt_fr, wlt_pat, wrsum, bsum, wh1t, bh1,
                    wh2, bh2, *out_and_scratch):
  out_refs = out_and_scratch[:-1]
  acc = out_and_scratch[-1]
  i = pl.program_id(0)
  et = pl.program_id(1)

  @pl.when(et == 0)
  def _():
    x = _concat_chunks((x0, x1, x2, x3))
    sp = _concat_chunks((p0, p1, p2, p3)) * _inv_counts(cpat[...])
    acc[...] = (
        jnp.dot(sp, wlt_pat[...], preferred_element_type=jnp.float32)
        + jnp.dot(x, wrsum[...], preferred_element_type=jnp.float32)
        + bsum[...])

  @pl.when(i < nlow)
  def _():
    sf = _concat_chunks((f0, f1, f2, f3)) * _inv_counts(cfr[...])
    acc[...] += jnp.dot(sf, wlt_fr[0], preferred_element_type=jnp.float32)

  @pl.when(et == 4)
  def _():
    y = _elu(acc[...])
    if head:
      h = jnp.maximum(
          jnp.dot(y, wh1t[...], preferred_element_type=jnp.float32)
          + bh1[...], 0.0)
      o = jnp.sum(h * wh2[...], axis=1) + bh2[0]
      out_refs[0][...] = o
    else:
      _chunk_writes(y, out_refs)


def _tx_update(head, s_fr, c_fr, s_pat, c_pat, x_tx, wlt_fr, wlt_pat, wrsum,
               bsum, wh1t, bh1, wh2, bh2):
  grid = (TX_PAD // BN, 5)
  nlow = ENT_PAD // BN
  fspec = pl.BlockSpec(
      (BN, CW), lambda i, et: (et * nlow + jnp.minimum(i, nlow - 1), 0))
  cfspec = pl.BlockSpec(
      (BN, 16), lambda i, et: (et * nlow + jnp.minimum(i, nlow - 1), 0))
  pspec = pl.BlockSpec((BN, CW), lambda i, et: (i, 0))
  if head:
    out_specs = [pl.BlockSpec((BN,), lambda i, et: (i,))]
    out_shape = [jax.ShapeDtypeStruct((TX_PAD,), jnp.float32)]
  else:
    out_specs = [pspec] * NCHUNK
    out_shape = [jax.ShapeDtypeStruct((TX_PAD, CW), jnp.float32)] * NCHUNK
  return pl.pallas_call(
      functools.partial(_tx_update_body, head, nlow),
      grid=grid,
      in_specs=(
          [fspec] * NCHUNK + [cfspec]
          + [pspec] * NCHUNK
          + [pl.BlockSpec((BN, 16), lambda i, et: (i, 0))]
          + [pspec] * NCHUNK
          + [pl.BlockSpec((1, H, H), lambda i, et: (et, 0, 0)),
             pl.BlockSpec((H, H), lambda i, et: (0, 0)),
             pl.BlockSpec((H, H), lambda i, et: (0, 0)),
             pl.BlockSpec((1, H), lambda i, et: (0, 0)),
             pl.BlockSpec((H, H), lambda i, et: (0, 0)),
             pl.BlockSpec((1, H), lambda i, et: (0, 0)),
             pl.BlockSpec((1, H), lambda i, et: (0, 0)),
             pl.BlockSpec((1,), lambda i, et: (0,))]
      ),
      out_specs=out_specs,
      out_shape=out_shape,
      scratch_shapes=[pltpu.VMEM((BN, H), jnp.float32)],
  )(*s_fr, c_fr, *s_pat, c_pat, *x_tx, wlt_fr, wlt_pat, wrsum, bsum,
    wh1t, bh1, wh2, bh2)


@functools.cache
def _mesh():
  return plsc.VectorSubcoreMesh(
      core_axis_name="c", subcore_axis_name="s", num_cores=NC,
      num_subcores=NS)


def _secs(nrow):
  out = []
  off = 0
  while off < nrow:
    sz = min(48, nrow - off)
    out.append((off, sz))
    off += sz
  return out


def _zero_rows_buf(rows):
  def body(i, _):
    for b in range(2):
      rows[b, i, pl.ds(0, 16)] = jnp.zeros((16,), jnp.float32)
      rows[b, i, pl.ds(16, 16)] = jnp.zeros((16,), jnp.float32)
    return 0

  lax.fori_loop(0, 128, body, 0)


def _zero_fill(zsrc, acc, rows, s):
  per_tile = rows // NS
  base = s * per_tile
  off = 0
  while off < per_tile:
    sz = min(128, per_tile - off)
    pltpu.sync_copy(zsrc.at[pl.ds(0, sz)], acc.at[pl.ds(base + off, sz)])
    off += sz


def _agg_phase(tab, src_hbm, dst_hbm, acc, rows_acc, isrc, idst, rows,
               sems, s):
  nrow = src_hbm.shape[1]
  _zero_rows_buf(rows)
  _zero_fill(rows.at[0], acc, rows_acc, s)
  plsc.subcore_barrier()
  for off, sz in _secs(nrow):
    pltpu.sync_copy(src_hbm.at[s, pl.ds(off, sz)], isrc.at[pl.ds(0, sz)])
    pltpu.sync_copy(dst_hbm.at[s, pl.ds(off, sz)], idst.at[pl.ds(0, sz)])
    pltpu.async_copy(tab.at[isrc.at[0]], rows.at[0], sems.at[0])

    def body(j, _):
      jm = j % 2
      jn = (j + 1) % 2

      @pl.when(j + 1 < sz)
      def _():
        pltpu.async_copy(tab.at[isrc.at[j + 1]], rows.at[jn], sems.at[jn])

      pltpu.make_async_copy(
          tab.at[isrc.at[j]], rows.at[jm], sems.at[jm]).wait()
      pltpu.sync_copy(rows.at[jm], acc.at[idst.at[j]], add=True)
      return 0

    lax.fori_loop(0, sz, body, 0)
  plsc.subcore_barrier()


@functools.cache
def _build_sc_layer(include_to):
  n_out = (3 if include_to else 2) * NCHUNK
  out_type = (
      ([jax.ShapeDtypeStruct((ENT_ALL, CW), jnp.float32)] * NCHUNK
       if include_to else [])
      + [jax.ShapeDtypeStruct((ENT_ALL, CW), jnp.float32)] * NCHUNK
      + [jax.ShapeDtypeStruct((TX_PAD, CW), jnp.float32)] * NCHUNK)

  def body(*refs):
    txc = refs[0:4]
    entc = refs[4:8]
    to_src, to_dst, fr_src, fr_dst, pat_src, pat_dst = refs[8:14]
    outs = refs[14:14 + n_out]
    if include_to:
      s_to = outs[0:4]
      s_fr = outs[4:8]
      s_pat = outs[8:12]
    else:
      s_fr = outs[0:4]
      s_pat = outs[4:8]
    acc, isrc, idst, rows, sems = refs[14 + n_out:]

    c = lax.axis_index("c")
    s = lax.axis_index("s")

    for f in range(NCHUNK):
      with_f = pl.when(c == f // 2)

      if include_to:
        @with_f
        def _(f=f):
          _agg_phase(txc[f], to_src, to_dst, acc, TO_ACC, isrc, idst,
                     rows, sems, s)
          per = ENT_ALL // NS
          pltpu.sync_copy(acc.at[pl.ds(s * per, per)],
                          s_to[f].at[pl.ds(s * per, per)])
          plsc.subcore_barrier()

      @with_f
      def _(f=f):
        _agg_phase(entc[f], fr_src, fr_dst, acc, FR_ACC, isrc, idst,
                   rows, sems, s)
        per = ENT_PAD // NS
        for et in range(5):
          pltpu.sync_copy(
              acc.at[pl.ds(et * FR_STRIDE + s * per, per)],
              s_fr[f].at[pl.ds(et * ENT_PAD + s * per, per)])
        plsc.subcore_barrier()

      @with_f
      def _(f=f):
        _agg_phase(txc[f], pat_src, pat_dst, acc, PAT_ACC, isrc, idst,
                   rows, sems, s)
        per = TX_PAD // NS
        pltpu.sync_copy(acc.at[pl.ds(s * per, per)],
                        s_pat[f].at[pl.ds(s * per, per)])
        plsc.subcore_barrier()

  return pl.kernel(
      body,
      out_type=out_type,
      mesh=_mesh(),
      compiler_params=pltpu.CompilerParams(use_tc_tiling_on_sc=False),
      scratch_types=[
          pltpu.VMEM_SHARED((FR_ACC, CW), jnp.float32),
          pltpu.VMEM((48, 128), jnp.int32),
          pltpu.VMEM((48, 128), jnp.int32),
          pltpu.VMEM((2, 128, CW), jnp.float32),
          pltpu.SemaphoreType.DMA((2,)),
      ],
  )


def _counts_phase(dst_hbm, cacc, rows_acc, ones, zeros, idst, s):
  nrow = dst_hbm.shape[1]
  per_tile = rows_acc // NS
  base = s * per_tile
  off = 0
  while off < per_tile:
    sz = min(128, per_tile - off)
    pltpu.sync_copy(zeros.at[pl.ds(0, sz)], cacc.at[pl.ds(base + off, sz)])
    off += sz
  plsc.subcore_barrier()
  for off, sz in _secs(nrow):
    pltpu.sync_copy(dst_hbm.at[s, pl.ds(off, sz)], idst.at[pl.ds(0, sz)])

    def body(j, _):
      pltpu.sync_copy(ones, cacc.at[idst.at[j]], add=True)
      return 0

    lax.fori_loop(0, sz, body, 0)
  plsc.subcore_barrier()


@functools.cache
def _build_sc_counts():
  out_type = [
      jax.ShapeDtypeStruct((ENT_ALL, 16), jnp.float32),
      jax.ShapeDtypeStruct((ENT_ALL, 16), jnp.float32),
      jax.ShapeDtypeStruct((TX_PAD, 16), jnp.float32),
  ]

  def body(to_dst, fr_dst, pat_dst, c_to, c_fr, c_pat, cacc, idst, ones,
           zeros):
    c = lax.axis_index("c")
    s = lax.axis_index("s")

    def fill(i, _):
      ones[i, :] = jnp.ones((16,), jnp.float32)
      zeros[i, :] = jnp.zeros((16,), jnp.float32)
      return 0

    lax.fori_loop(0, 128, fill, 0)
    plsc.subcore_barrier()

    @pl.when(c == 0)
    def _():
      _counts_phase(to_dst, cacc, TO_ACC, ones, zeros, idst, s)
      per = ENT_ALL // NS
      pltpu.sync_copy(cacc.at[pl.ds(s * per, per)],
                      c_to.at[pl.ds(s * per, per)])
      plsc.subcore_barrier()

    @pl.when(c == 1)
    def _():
      _counts_phase(fr_dst, cacc, FR_ACC, ones, zeros, idst, s)
      per = ENT_PAD // NS
      for et in range(5):
        pltpu.sync_copy(
            cacc.at[pl.ds(et * FR_STRIDE + s * per, per)],
            c_fr.at[pl.ds(et * ENT_PAD + s * per, per)])
      plsc.subcore_barrier()
      _counts_phase(pat_dst, cacc, PAT_ACC, ones, zeros, idst, s)
      per = TX_PAD // NS
      pltpu.sync_copy(cacc.at[pl.ds(s * per, per)],
                      c_pat.at[pl.ds(s * per, per)])
      plsc.subcore_barrier()

  return pl.kernel(
      body,
      out_type=out_type,
      mesh=_mesh(),
      compiler_params=pltpu.CompilerParams(use_tc_tiling_on_sc=False),
      scratch_types=[
          pltpu.VMEM_SHARED((FR_ACC, 16), jnp.float32),
          pltpu.VMEM((48, 128), jnp.int32),
          pltpu.VMEM((128, 16), jnp.float32),
          pltpu.VMEM((128, 16), jnp.float32),
      ],
  )


def _pad_rows(x, n):
  return jnp.pad(x, ((0, n - x.shape[0]),) + ((0, 0),) * (x.ndim - 1))


def _edges(eis, offs_dst, dump, total, offs_src=None):
  srcs, dsts = [], []
  for k, ei in enumerate(eis):
    s = ei[0].astype(jnp.int32)
    if offs_src is not None:
      s = s + offs_src[k]
    srcs.append(s)
    dsts.append(ei[1].astype(jnp.int32) + offs_dst[k])
  ns = jnp.concatenate(srcs) if len(srcs) > 1 else srcs[0]
  nd = jnp.concatenate(dsts) if len(dsts) > 1 else dsts[0]
  pad = total - ns.shape[0]
  ns = jnp.concatenate([ns, jnp.zeros((pad,), jnp.int32)])
  nd = jnp.concatenate([nd, jnp.full((pad,), dump, jnp.int32)])
  return ns.reshape(NS, -1, 128), nd.reshape(NS, -1, 128)


def kernel(tx_x, card_x, addr_x, email_x, device_x, product_x, ei_to_card,
           ei_from_card, ei_to_addr, ei_from_addr, ei_to_email, ei_from_email,
           ei_to_device, ei_from_device, ei_to_product, ei_from_product,
           ei_tx_pattern, params):
  ents = [card_x, addr_x, email_x, device_x, product_x]
  ei_to = [ei_to_card, ei_to_addr, ei_to_email, ei_to_device, ei_to_product]
  ei_fr = [ei_from_card, ei_from_addr, ei_from_email, ei_from_device,
           ei_from_product]

  tx_pad = _pad_rows(tx_x, TX_PAD)
  ent_all = jnp.concatenate([_pad_rows(e, ENT_PAD) for e in ents])

  to_src, to_dst = _edges(
      ei_to, [et * ENT_PAD for et in range(5)], TO_DUMP, TO_E)
  fr_src, fr_dst = _edges(
      ei_fr, [et * FR_STRIDE for et in range(5)], FR_DUMP, FR_E,
      offs_src=[et * ENT_PAD for et in range(5)])
  pat_src, pat_dst = _edges([ei_tx_pattern], [0], PAT_DUMP, PAT_E)

  p = params
  w1t, b1 = p["tx_proj"][0].T, p["tx_proj"][1][None, :]
  w2t, b2 = p["tx_proj"][2].T, p["tx_proj"][3][None, :]
  wet, be = p["entity_proj"][0].T, p["entity_proj"][1][None, :]
  wh1t, bh1 = p["head"][0].T, p["head"][1][None, :]
  wh2, bh2 = p["head"][2], p["head"][3]

  ets = ["card", "addr", "email", "device", "product"]
  layer_w = []
  for layer in p["layers"]:
    wlt_to = jnp.stack([layer["to_" + et][0].T for et in ets])
    bl_to = jnp.stack([layer["to_" + et][1] for et in ets])[:, None, :]
    wrt_to = jnp.stack([layer["to_" + et][2].T for et in ets])
    wlt_fr = jnp.stack([layer["from_" + et][0].T / 6.0 for et in ets])
    wlt_pat = layer["tx_pattern"][0].T / 6.0
    wr_sum = sum(layer["from_" + et][2] for et in ets) + \
        layer["tx_pattern"][2]
    wrsum_t = wr_sum.T / 6.0
    bsum = (sum(layer["from_" + et][1] for et in ets)
            + layer["tx_pattern"][1])[None, :] / 6.0
    layer_w.append((wlt_to, bl_to, wrt_to, wlt_fr, wlt_pat, wrsum_t, bsum))

  x_tx = _tx_proj(tx_pad, w1t, b1, w2t, b2)
  x_ent = _ent_proj(ent_all, wet, be)

  c_to, c_fr, c_pat = _build_sc_counts()(to_dst, fr_dst, pat_dst)

  dummy = jnp.zeros((1, H), jnp.float32)
  dummy1 = jnp.zeros((1,), jnp.float32)

  wlt_to, bl_to, wrt_to, wlt_fr, wlt_pat, wrsum_t, bsum = layer_w[0]
  outs = _build_sc_layer(True)(*x_tx, *x_ent, to_src, to_dst, fr_src, fr_dst,
                    pat_src, pat_dst)
  s_to, s_fr, s_pat = outs[0:4], outs[4:8], outs[8:12]
  x_ent1 = _ent_update(s_to, c_to, x_ent, wlt_to, bl_to, wrt_to)
  x_tx1 = _tx_update(False, s_fr, c_fr, s_pat, c_pat, x_tx, wlt_fr, wlt_pat,
                     wrsum_t, bsum, dummy, dummy, dummy, dummy1)

  _, _, _, wlt_fr, wlt_pat, wrsum_t, bsum = layer_w[1]
  outs = _build_sc_layer(False)(*x_tx1, *x_ent1, to_src, to_dst, fr_src, fr_dst,
                    pat_src, pat_dst)
  s_fr, s_pat = outs[0:4], outs[4:8]
  (out,) = _tx_update(True, s_fr, c_fr, s_pat, c_pat, x_tx1, wlt_fr, wlt_pat,
                      wrsum_t, bsum, wh1t, bh1, wh2, bh2)
  return out[:N_TX]


# speedup vs baseline: 2.9293x; 1.3129x over previous
import functools
import jax
import jax.numpy as jnp
from jax import lax
from jax.experimental import pallas as pl
from jax.experimental.pallas import tpu as pltpu
from jax.experimental.pallas import tpu_sc as plsc

N_TX = 50000
N_ENT = 10000
H = 128
NCHUNK = 4
CW = 32
NS = 16
NC = 2

TX_PAD = 50176
ENT_PAD = 10240
ENT_ALL = 5 * ENT_PAD

TO_ACC = 51328
FR_STRIDE = 10248
FR_ACC = 51328
PAT_ACC = 50304

TO_DUMP = TO_ACC - 1
FR_DUMP = FR_ACC - 1
PAT_DUMP = PAT_ACC - 1

TO_E = 251904
FR_E = 251904
PAT_E = 100352

BN = 512


def _elu(x):
  return jnp.where(x > 0, x, jnp.exp(x) - 1.0)


def _inv_counts(cblk):
  c = cblk[:, 0:1]
  return 1.0 / jnp.maximum(c, 1.0)


def _chunk_writes(y, out_refs):
  for f in range(NCHUNK):
    out_refs[f][...] = y[:, f * CW:(f + 1) * CW]


def _concat_chunks(refs):
  return jnp.concatenate([refs[f][...] for f in range(NCHUNK)], axis=1)


def _tx_proj_body(x_ref, w1_ref, b1_ref, w2_ref, b2_ref, *out_refs):
  h = jnp.maximum(
      jnp.dot(x_ref[...], w1_ref[...], preferred_element_type=jnp.float32)
      + b1_ref[...], 0.0)
  y = _elu(
      jnp.dot(h, w2_ref[...], preferred_element_type=jnp.float32)
      + b2_ref[...])
  _chunk_writes(y, out_refs)


def _tx_proj(tx_pad, w1t, b1, w2t, b2):
  grid = TX_PAD // BN
  return pl.pallas_call(
      _tx_proj_body,
      grid=(grid,),
      in_specs=[
          pl.BlockSpec((BN, H), lambda i: (i, 0)),
          pl.BlockSpec((H, H), lambda i: (0, 0)),
          pl.BlockSpec((1, H), lambda i: (0, 0)),
          pl.BlockSpec((H, H), lambda i: (0, 0)),
          pl.BlockSpec((1, H), lambda i: (0, 0)),
      ],
      out_specs=[pl.BlockSpec((BN, CW), lambda i: (i, 0))] * NCHUNK,
      out_shape=[jax.ShapeDtypeStruct((TX_PAD, CW), jnp.float32)] * NCHUNK,
  )(tx_pad, w1t, b1, w2t, b2)


def _ent_proj_body(x_ref, we_ref, be_ref, *out_refs):
  y = jnp.maximum(
      jnp.dot(x_ref[...], we_ref[...], preferred_element_type=jnp.float32)
      + be_ref[...], 0.0)
  _chunk_writes(y, out_refs)


def _ent_proj(ent_all, wet, be):
  grid = ENT_ALL // BN
  return pl.pallas_call(
      _ent_proj_body,
      grid=(grid,),
      in_specs=[
          pl.BlockSpec((BN, 2), lambda i: (i, 0)),
          pl.BlockSpec((2, H), lambda i: (0, 0)),
          pl.BlockSpec((1, H), lambda i: (0, 0)),
      ],
      out_specs=[pl.BlockSpec((BN, CW), lambda i: (i, 0))] * NCHUNK,
      out_shape=[jax.ShapeDtypeStruct((ENT_ALL, CW), jnp.float32)] * NCHUNK,
  )(ent_all, wet, be)


def _ent_update_body(s0, s1, s2, s3, cto, x0, x1, x2, x3, wlt, bl, wrt,
                     *out_refs):
  agg = _concat_chunks((s0, s1, s2, s3)) * _inv_counts(cto[...])
  x = _concat_chunks((x0, x1, x2, x3))
  y = _elu(
      jnp.dot(agg, wlt[0], preferred_element_type=jnp.float32)
      + bl[0]
      + jnp.dot(x, wrt[0], preferred_element_type=jnp.float32))
  _chunk_writes(y, out_refs)


def _ent_update(s_to, c_to, x_ent, wlt_to, bl_to, wrt_to):
  grid = ENT_ALL // BN
  blocks_per_et = ENT_PAD // BN
  sspec = pl.BlockSpec((BN, CW), lambda i: (i, 0))
  return pl.pallas_call(
      _ent_update_body,
      grid=(grid,),
      in_specs=(
          [sspec] * NCHUNK
          + [pl.BlockSpec((BN, 16), lambda i: (i, 0))]
          + [sspec] * NCHUNK
          + [pl.BlockSpec((1, H, H), lambda i: (i // blocks_per_et, 0, 0)),
             pl.BlockSpec((1, 1, H), lambda i: (i // blocks_per_et, 0, 0)),
             pl.BlockSpec((1, H, H), lambda i: (i // blocks_per_et, 0, 0))]
      ),
      out_specs=[sspec] * NCHUNK,
      out_shape=[jax.ShapeDtypeStruct((ENT_ALL, CW), jnp.float32)] * NCHUNK,
  )(*s_to, c_to, *x_ent, wlt_to, bl_to, wrt_to)


def _tx_update_body(head, nlow, *refs):
  frs = refs[0:20]
  cfrs = refs[20:25]
  ps = refs[25:29]
  cpat = refs[29]
  xs = refs[30:34]
  (wlt_fr, wlt_pat, wrsum, bsum, wh1t, bh1, wh2, bh2) = refs[34:42]
  out_refs = refs[42:]
  i = pl.program_id(0)

  x = _concat_chunks(xs)
  sp = _concat_chunks(ps) * _inv_counts(cpat[...])
  acc = (jnp.dot(sp, wlt_pat[...], preferred_element_type=jnp.float32)
         + jnp.dot(x, wrsum[...], preferred_element_type=jnp.float32)
         + bsum[...])

  @pl.when(i < nlow)
  def _():
    z = acc
    for et in range(5):
      sf = _concat_chunks(frs[4 * et:4 * et + 4]) * _inv_counts(
          cfrs[et][...])
      z = z + jnp.dot(sf, wlt_fr[et],
                      preferred_element_type=jnp.float32)
    _tx_update_out(head, z, out_refs, wh1t, bh1, wh2, bh2)

  @pl.when(i >= nlow)
  def _():
    _tx_update_out(head, acc, out_refs, wh1t, bh1, wh2, bh2)


def _tx_update_out(head, z, out_refs, wh1t, bh1, wh2, bh2):
  y = _elu(z)
  if head:
    h = jnp.maximum(
        jnp.dot(y, wh1t[...], preferred_element_type=jnp.float32)
        + bh1[...], 0.0)
    o = jnp.sum(h * wh2[...], axis=1) + bh2[0]
    out_refs[0][...] = o
  else:
    _chunk_writes(y, out_refs)


def _tx_update(head, s_fr, c_fr, s_pat, c_pat, x_tx, wlt_fr, wlt_pat, wrsum,
               bsum, wh1t, bh1, wh2, bh2):
  grid = (TX_PAD // BN,)
  nlow = ENT_PAD // BN
  fspecs, cfspecs = [], []
  for et in range(5):
    fspecs += [pl.BlockSpec(
        (BN, CW),
        lambda i, et=et: (et * nlow + jnp.minimum(i, nlow - 1), 0))] * NCHUNK
    cfspecs += [pl.BlockSpec(
        (BN, 16),
        lambda i, et=et: (et * nlow + jnp.minimum(i, nlow - 1), 0))]
  pspec = pl.BlockSpec((BN, CW), lambda i: (i, 0))
  if head:
    out_specs = [pl.BlockSpec((BN,), lambda i: (i,))]
    out_shape = [jax.ShapeDtypeStruct((TX_PAD,), jnp.float32)]
  else:
    out_specs = [pspec] * NCHUNK
    out_shape = [jax.ShapeDtypeStruct((TX_PAD, CW), jnp.float32)] * NCHUNK
  return pl.pallas_call(
      functools.partial(_tx_update_body, head, nlow),
      grid=grid,
      in_specs=(
          fspecs + cfspecs
          + [pspec] * NCHUNK
          + [pl.BlockSpec((BN, 16), lambda i: (i, 0))]
          + [pspec] * NCHUNK
          + [pl.BlockSpec((5, H, H), lambda i: (0, 0, 0)),
             pl.BlockSpec((H, H), lambda i: (0, 0)),
             pl.BlockSpec((H, H), lambda i: (0, 0)),
             pl.BlockSpec((1, H), lambda i: (0, 0)),
             pl.BlockSpec((H, H), lambda i: (0, 0)),
             pl.BlockSpec((1, H), lambda i: (0, 0)),
             pl.BlockSpec((1, H), lambda i: (0, 0)),
             pl.BlockSpec((1,), lambda i: (0,))]
      ),
      out_specs=out_specs,
      out_shape=out_shape,
  )(*(list(s_fr) * 5), *([c_fr] * 5), *s_pat, c_pat, *x_tx,
    wlt_fr, wlt_pat, wrsum, bsum, wh1t, bh1, wh2, bh2)


@functools.cache
def _mesh():
  return plsc.VectorSubcoreMesh(
      core_axis_name="c", subcore_axis_name="s", num_cores=NC,
      num_subcores=NS)


def _secs(nrow, width=48):
  out = []
  off = 0
  while off < nrow:
    sz = min(width, nrow - off)
    out.append((off, sz))
    off += sz
  return out


def _zero_rows_buf(rows):
  def body(i, _):
    for b in range(4):
      rows[b, i, pl.ds(0, 16)] = jnp.zeros((16,), jnp.float32)
      rows[b, i, pl.ds(16, 16)] = jnp.zeros((16,), jnp.float32)
    return 0

  lax.fori_loop(0, 128, body, 0)


def _zero_fill(zsrc, acc, rows, s):
  per_tile = rows // NS
  base = s * per_tile
  off = 0
  while off < per_tile:
    sz = min(128, per_tile - off)
    pltpu.sync_copy(zsrc.at[pl.ds(0, sz)], acc.at[pl.ds(base + off, sz)])
    off += sz


def _agg_phase(tab, src_hbm, dst_hbm, acc, rows_acc, isrc, idst, rows,
               gsems, ssems, s):
  nrow = src_hbm.shape[1]
  _zero_rows_buf(rows)
  _zero_fill(rows.at[0], acc, rows_acc, s)
  plsc.subcore_barrier()
  for off, sz in _secs(nrow, 24):
    pltpu.sync_copy(src_hbm.at[s, pl.ds(off, sz)], isrc.at[pl.ds(0, sz)])
    pltpu.sync_copy(dst_hbm.at[s, pl.ds(off, sz)], idst.at[pl.ds(0, sz)])
    for j in range(min(2, sz)):
      pltpu.async_copy(tab.at[isrc.at[j]], rows.at[j], gsems.at[j])

    def body(j, _):
      b = j % 4
      bn = (j + 2) % 4

      @pl.when(j + 2 < sz)
      def _():
        @pl.when(j >= 2)
        def _():
          pltpu.make_async_copy(
              rows.at[bn], acc.at[idst.at[j - 2]], ssems.at[bn]).wait()

        pltpu.async_copy(tab.at[isrc.at[j + 2]], rows.at[bn], gsems.at[bn])

      pltpu.make_async_copy(
          tab.at[isrc.at[j]], rows.at[b], gsems.at[b]).wait()
      pltpu.async_copy(rows.at[b], acc.at[idst.at[j]], ssems.at[b], add=True)
      return 0

    lax.fori_loop(0, sz, body, 0)
    for k in range(max(0, sz - 4), sz):
      pltpu.make_async_copy(
          rows.at[k % 4], acc.at[idst.at[k]], ssems.at[k % 4]).wait()
  plsc.subcore_barrier()


@functools.cache
def _build_sc_layer(include_to):
  n_out = (3 if include_to else 2) * NCHUNK
  out_type = (
      ([jax.ShapeDtypeStruct((ENT_ALL, CW), jnp.float32)] * NCHUNK
       if include_to else [])
      + [jax.ShapeDtypeStruct((ENT_ALL, CW), jnp.float32)] * NCHUNK
      + [jax.ShapeDtypeStruct((TX_PAD, CW), jnp.float32)] * NCHUNK)

  def body(*refs):
    txc = refs[0:4]
    entc = refs[4:8]
    to_src, to_dst, fr_src, fr_dst, pat_src, pat_dst = refs[8:14]
    outs = refs[14:14 + n_out]
    if include_to:
      s_to = outs[0:4]
      s_fr = outs[4:8]
      s_pat = outs[8:12]
    else:
      s_fr = outs[0:4]
      s_pat = outs[4:8]
    acc, isrc, idst, rows, gsems, ssems = refs[14 + n_out:]

    c = lax.axis_index("c")
    s = lax.axis_index("s")

    for f in range(NCHUNK):
      with_f = pl.when(c == f // 2)

      if include_to:
        @with_f
        def _(f=f):
          _agg_phase(txc[f], to_src, to_dst, acc, TO_ACC, isrc, idst,
                     rows, gsems, ssems, s)
          per = ENT_ALL // NS
          pltpu.sync_copy(acc.at[pl.ds(s * per, per)],
                          s_to[f].at[pl.ds(s * per, per)])
          plsc.subcore_barrier()

      @with_f
      def _(f=f):
        _agg_phase(entc[f], fr_src, fr_dst, acc, FR_ACC, isrc, idst,
                   rows, gsems, ssems, s)
        per = ENT_PAD // NS
        for et in range(5):
          pltpu.sync_copy(
              acc.at[pl.ds(et * FR_STRIDE + s * per, per)],
              s_fr[f].at[pl.ds(et * ENT_PAD + s * per, per)])
        plsc.subcore_barrier()

      @with_f
      def _(f=f):
        _agg_phase(txc[f], pat_src, pat_dst, acc, PAT_ACC, isrc, idst,
                   rows, gsems, ssems, s)
        per = TX_PAD // NS
        pltpu.sync_copy(acc.at[pl.ds(s * per, per)],
                        s_pat[f].at[pl.ds(s * per, per)])
        plsc.subcore_barrier()

  return pl.kernel(
      body,
      out_type=out_type,
      mesh=_mesh(),
      compiler_params=pltpu.CompilerParams(use_tc_tiling_on_sc=False),
      scratch_types=[
          pltpu.VMEM_SHARED((FR_ACC, CW), jnp.float32),
          pltpu.VMEM((24, 128), jnp.int32),
          pltpu.VMEM((24, 128), jnp.int32),
          pltpu.VMEM((4, 128, CW), jnp.float32),
          pltpu.SemaphoreType.DMA((4,)),
          pltpu.SemaphoreType.DMA((4,)),
      ],
  )


def _counts_phase(dst_hbm, cacc, rows_acc, ones, zeros, idst, s):
  nrow = dst_hbm.shape[1]
  per_tile = rows_acc // NS
  base = s * per_tile
  off = 0
  while off < per_tile:
    sz = min(128, per_tile - off)
    pltpu.sync_copy(zeros.at[pl.ds(0, sz)], cacc.at[pl.ds(base + off, sz)])
    off += sz
  plsc.subcore_barrier()
  for off, sz in _secs(nrow):
    pltpu.sync_copy(dst_hbm.at[s, pl.ds(off, sz)], idst.at[pl.ds(0, sz)])

    def body(j, _):
      pltpu.sync_copy(ones, cacc.at[idst.at[j]], add=True)
      return 0

    lax.fori_loop(0, sz, body, 0)
  plsc.subcore_barrier()


@functools.cache
def _build_sc_counts():
  out_type = [
      jax.ShapeDtypeStruct((ENT_ALL, 16), jnp.float32),
      jax.ShapeDtypeStruct((ENT_ALL, 16), jnp.float32),
      jax.ShapeDtypeStruct((TX_PAD, 16), jnp.float32),
  ]

  def body(to_dst, fr_dst, pat_dst, c_to, c_fr, c_pat, cacc, idst, ones,
           zeros):
    c = lax.axis_index("c")
    s = lax.axis_index("s")

    def fill(i, _):
      ones[i, :] = jnp.ones((16,), jnp.float32)
      zeros[i, :] = jnp.zeros((16,), jnp.float32)
      return 0

    lax.fori_loop(0, 128, fill, 0)
    plsc.subcore_barrier()

    @pl.when(c == 0)
    def _():
      _counts_phase(to_dst, cacc, TO_ACC, ones, zeros, idst, s)
      per = ENT_ALL // NS
      pltpu.sync_copy(cacc.at[pl.ds(s * per, per)],
                      c_to.at[pl.ds(s * per, per)])
      plsc.subcore_barrier()

    @pl.when(c == 1)
    def _():
      _counts_phase(fr_dst, cacc, FR_ACC, ones, zeros, idst, s)
      per = ENT_PAD // NS
      for et in range(5):
        pltpu.sync_copy(
            cacc.at[pl.ds(et * FR_STRIDE + s * per, per)],
            c_fr.at[pl.ds(et * ENT_PAD + s * per, per)])
      plsc.subcore_barrier()
      _counts_phase(pat_dst, cacc, PAT_ACC, ones, zeros, idst, s)
      per = TX_PAD // NS
      pltpu.sync_copy(cacc.at[pl.ds(s * per, per)],
                      c_pat.at[pl.ds(s * per, per)])
      plsc.subcore_barrier()

  return pl.kernel(
      body,
      out_type=out_type,
      mesh=_mesh(),
      compiler_params=pltpu.CompilerParams(use_tc_tiling_on_sc=False),
      scratch_types=[
          pltpu.VMEM_SHARED((FR_ACC, 16), jnp.float32),
          pltpu.VMEM((48, 128), jnp.int32),
          pltpu.VMEM((128, 16), jnp.float32),
          pltpu.VMEM((128, 16), jnp.float32),
      ],
  )


def _pad_rows(x, n):
  return jnp.pad(x, ((0, n - x.shape[0]),) + ((0, 0),) * (x.ndim - 1))


def _edges(eis, offs_dst, dump, total, offs_src=None):
  srcs, dsts = [], []
  for k, ei in enumerate(eis):
    s = ei[0].astype(jnp.int32)
    if offs_src is not None:
      s = s + offs_src[k]
    srcs.append(s)
    dsts.append(ei[1].astype(jnp.int32) + offs_dst[k])
  ns = jnp.concatenate(srcs) if len(srcs) > 1 else srcs[0]
  nd = jnp.concatenate(dsts) if len(dsts) > 1 else dsts[0]
  pad = total - ns.shape[0]
  ns = jnp.concatenate([ns, jnp.zeros((pad,), jnp.int32)])
  nd = jnp.concatenate([nd, jnp.full((pad,), dump, jnp.int32)])
  return ns.reshape(NS, -1, 128), nd.reshape(NS, -1, 128)


def kernel(tx_x, card_x, addr_x, email_x, device_x, product_x, ei_to_card,
           ei_from_card, ei_to_addr, ei_from_addr, ei_to_email, ei_from_email,
           ei_to_device, ei_from_device, ei_to_product, ei_from_product,
           ei_tx_pattern, params):
  ents = [card_x, addr_x, email_x, device_x, product_x]
  ei_to = [ei_to_card, ei_to_addr, ei_to_email, ei_to_device, ei_to_product]
  ei_fr = [ei_from_card, ei_from_addr, ei_from_email, ei_from_device,
           ei_from_product]

  tx_pad = _pad_rows(tx_x, TX_PAD)
  ent_all = jnp.concatenate([_pad_rows(e, ENT_PAD) for e in ents])

  to_src, to_dst = _edges(
      ei_to, [et * ENT_PAD for et in range(5)], TO_DUMP, TO_E)
  fr_src, fr_dst = _edges(
      ei_fr, [et * FR_STRIDE for et in range(5)], FR_DUMP, FR_E,
      offs_src=[et * ENT_PAD for et in range(5)])
  pat_src, pat_dst = _edges([ei_tx_pattern], [0], PAT_DUMP, PAT_E)

  p = params
  w1t, b1 = p["tx_proj"][0].T, p["tx_proj"][1][None, :]
  w2t, b2 = p["tx_proj"][2].T, p["tx_proj"][3][None, :]
  wet, be = p["entity_proj"][0].T, p["entity_proj"][1][None, :]
  wh1t, bh1 = p["head"][0].T, p["head"][1][None, :]
  wh2, bh2 = p["head"][2], p["head"][3]

  ets = ["card", "addr", "email", "device", "product"]
  layer_w = []
  for layer in p["layers"]:
    wlt_to = jnp.stack([layer["to_" + et][0].T for et in ets])
    bl_to = jnp.stack([layer["to_" + et][1] for et in ets])[:, None, :]
    wrt_to = jnp.stack([layer["to_" + et][2].T for et in ets])
    wlt_fr = jnp.stack([layer["from_" + et][0].T / 6.0 for et in ets])
    wlt_pat = layer["tx_pattern"][0].T / 6.0
    wr_sum = sum(layer["from_" + et][2] for et in ets) + \
        layer["tx_pattern"][2]
    wrsum_t = wr_sum.T / 6.0
    bsum = (sum(layer["from_" + et][1] for et in ets)
            + layer["tx_pattern"][1])[None, :] / 6.0
    layer_w.append((wlt_to, bl_to, wrt_to, wlt_fr, wlt_pat, wrsum_t, bsum))

  x_tx = _tx_proj(tx_pad, w1t, b1, w2t, b2)
  x_ent = _ent_proj(ent_all, wet, be)

  c_to, c_fr, c_pat = _build_sc_counts()(to_dst, fr_dst, pat_dst)

  dummy = jnp.zeros((1, H), jnp.float32)
  dummy1 = jnp.zeros((1,), jnp.float32)

  wlt_to, bl_to, wrt_to, wlt_fr, wlt_pat, wrsum_t, bsum = layer_w[0]
  outs = _build_sc_layer(True)(*x_tx, *x_ent, to_src, to_dst, fr_src, fr_dst,
                    pat_src, pat_dst)
  s_to, s_fr, s_pat = outs[0:4], outs[4:8], outs[8:12]
  x_ent1 = _ent_update(s_to, c_to, x_ent, wlt_to, bl_to, wrt_to)
  x_tx1 = _tx_update(False, s_fr, c_fr, s_pat, c_pat, x_tx, wlt_fr, wlt_pat,
                     wrsum_t, bsum, dummy, dummy, dummy, dummy1)

  _, _, _, wlt_fr, wlt_pat, wrsum_t, bsum = layer_w[1]
  outs = _build_sc_layer(False)(*x_tx1, *x_ent1, to_src, to_dst, fr_src, fr_dst,
                    pat_src, pat_dst)
  s_fr, s_pat = outs[0:4], outs[4:8]
  (out,) = _tx_update(True, s_fr, c_fr, s_pat, c_pat, x_tx1, wlt_fr, wlt_pat,
                      wrsum_t, bsum, wh1t, bh1, wh2, bh2)
  return out[:N_TX]


# speedup vs baseline: 3.0747x; 1.0496x over previous
import functools
import jax
import jax.numpy as jnp
from jax import lax
from jax.experimental import pallas as pl
from jax.experimental.pallas import tpu as pltpu
from jax.experimental.pallas import tpu_sc as plsc

N_TX = 50000
N_ENT = 10000
H = 128
NCHUNK = 4
CW = 32
NS = 16
NC = 2

TX_PAD = 50176
ENT_PAD = 10240
ENT_ALL = 5 * ENT_PAD

TO_ACC = 51328
FR_STRIDE = 10248
FR_ACC = 51328
PAT_ACC = 50304

TO_DUMP = TO_ACC - 1
FR_DUMP = FR_ACC - 1
PAT_DUMP = PAT_ACC - 1

TO_E = 251904
FR_E = 251904
PAT_E = 100352

BN = 512


def _elu(x):
  return jnp.where(x > 0, x, jnp.exp(x) - 1.0)


def _inv_counts(cblk):
  c = cblk[:, 0:1]
  return 1.0 / jnp.maximum(c, 1.0)


def _chunk_writes(y, out_refs):
  for f in range(NCHUNK):
    out_refs[f][...] = y[:, f * CW:(f + 1) * CW]


def _concat_chunks(refs):
  return jnp.concatenate([refs[f][...] for f in range(NCHUNK)], axis=1)


def _tx_proj_body(x_ref, w1_ref, b1_ref, w2_ref, b2_ref, *out_refs):
  h = jnp.maximum(
      jnp.dot(x_ref[...], w1_ref[...], preferred_element_type=jnp.float32)
      + b1_ref[...], 0.0)
  y = _elu(
      jnp.dot(h, w2_ref[...], preferred_element_type=jnp.float32)
      + b2_ref[...])
  _chunk_writes(y, out_refs)


def _tx_proj(tx_pad, w1t, b1, w2t, b2):
  grid = TX_PAD // BN
  return pl.pallas_call(
      _tx_proj_body,
      grid=(grid,),
      in_specs=[
          pl.BlockSpec((BN, H), lambda i: (i, 0)),
          pl.BlockSpec((H, H), lambda i: (0, 0)),
          pl.BlockSpec((1, H), lambda i: (0, 0)),
          pl.BlockSpec((H, H), lambda i: (0, 0)),
          pl.BlockSpec((1, H), lambda i: (0, 0)),
      ],
      out_specs=[pl.BlockSpec((BN, CW), lambda i: (i, 0))] * NCHUNK,
      out_shape=[jax.ShapeDtypeStruct((TX_PAD, CW), jnp.float32)] * NCHUNK,
  )(tx_pad, w1t, b1, w2t, b2)


def _ent_proj_body(x_ref, we_ref, be_ref, *out_refs):
  y = jnp.maximum(
      jnp.dot(x_ref[...], we_ref[...], preferred_element_type=jnp.float32)
      + be_ref[...], 0.0)
  _chunk_writes(y, out_refs)


def _ent_proj(ent_all, wet, be):
  bne = 2048
  grid = ENT_ALL // bne
  return pl.pallas_call(
      _ent_proj_body,
      grid=(grid,),
      in_specs=[
          pl.BlockSpec((bne, 2), lambda i: (i, 0)),
          pl.BlockSpec((2, H), lambda i: (0, 0)),
          pl.BlockSpec((1, H), lambda i: (0, 0)),
      ],
      out_specs=[pl.BlockSpec((bne, CW), lambda i: (i, 0))] * NCHUNK,
      out_shape=[jax.ShapeDtypeStruct((ENT_ALL, CW), jnp.float32)] * NCHUNK,
  )(ent_all, wet, be)


def _ent_update_body(s0, s1, s2, s3, cto, x0, x1, x2, x3, wlt, bl, wrt,
                     *out_refs):
  agg = _concat_chunks((s0, s1, s2, s3)) * _inv_counts(cto[...])
  x = _concat_chunks((x0, x1, x2, x3))
  y = _elu(
      jnp.dot(agg, wlt[0], preferred_element_type=jnp.float32)
      + bl[0]
      + jnp.dot(x, wrt[0], preferred_element_type=jnp.float32))
  _chunk_writes(y, out_refs)


def _ent_update(s_to, c_to, x_ent, wlt_to, bl_to, wrt_to):
  grid = ENT_ALL // BN
  blocks_per_et = ENT_PAD // BN
  sspec = pl.BlockSpec((BN, CW), lambda i: (i, 0))
  return pl.pallas_call(
      _ent_update_body,
      grid=(grid,),
      in_specs=(
          [sspec] * NCHUNK
          + [pl.BlockSpec((BN, 16), lambda i: (i, 0))]
          + [sspec] * NCHUNK
          + [pl.BlockSpec((1, H, H), lambda i: (i // blocks_per_et, 0, 0)),
             pl.BlockSpec((1, 1, H), lambda i: (i // blocks_per_et, 0, 0)),
             pl.BlockSpec((1, H, H), lambda i: (i // blocks_per_et, 0, 0))]
      ),
      out_specs=[sspec] * NCHUNK,
      out_shape=[jax.ShapeDtypeStruct((ENT_ALL, CW), jnp.float32)] * NCHUNK,
  )(*s_to, c_to, *x_ent, wlt_to, bl_to, wrt_to)


def _tx_update_body(head, nlow, *refs):
  frs = refs[0:20]
  cfrs = refs[20:25]
  ps = refs[25:29]
  cpat = refs[29]
  xs = refs[30:34]
  (wlt_fr, wlt_pat, wrsum, bsum, wh1t, bh1, wh2, bh2) = refs[34:42]
  out_refs = refs[42:]
  i = pl.program_id(0)

  x = _concat_chunks(xs)
  sp = _concat_chunks(ps) * _inv_counts(cpat[...])
  acc = (jnp.dot(sp, wlt_pat[...], preferred_element_type=jnp.float32)
         + jnp.dot(x, wrsum[...], preferred_element_type=jnp.float32)
         + bsum[...])

  @pl.when(i < nlow)
  def _():
    z = acc
    for et in range(5):
      sf = _concat_chunks(frs[4 * et:4 * et + 4]) * _inv_counts(
          cfrs[et][...])
      z = z + jnp.dot(sf, wlt_fr[et],
                      preferred_element_type=jnp.float32)
    _tx_update_out(head, z, out_refs, wh1t, bh1, wh2, bh2)

  @pl.when(i >= nlow)
  def _():
    _tx_update_out(head, acc, out_refs, wh1t, bh1, wh2, bh2)


def _tx_update_out(head, z, out_refs, wh1t, bh1, wh2, bh2):
  y = _elu(z)
  if head:
    h = jnp.maximum(
        jnp.dot(y, wh1t[...], preferred_element_type=jnp.float32)
        + bh1[...], 0.0)
    o = jnp.sum(h * wh2[...], axis=1) + bh2[0]
    out_refs[0][...] = o
  else:
    _chunk_writes(y, out_refs)


def _tx_update(head, s_fr, c_fr, s_pat, c_pat, x_tx, wlt_fr, wlt_pat, wrsum,
               bsum, wh1t, bh1, wh2, bh2):
  grid = (TX_PAD // BN,)
  nlow = ENT_PAD // BN
  fspecs, cfspecs = [], []
  for et in range(5):
    fspecs += [pl.BlockSpec(
        (BN, CW),
        lambda i, et=et: (et * nlow + jnp.minimum(i, nlow - 1), 0))] * NCHUNK
    cfspecs += [pl.BlockSpec(
        (BN, 16),
        lambda i, et=et: (et * nlow + jnp.minimum(i, nlow - 1), 0))]
  pspec = pl.BlockSpec((BN, CW), lambda i: (i, 0))
  if head:
    out_specs = [pl.BlockSpec((BN,), lambda i: (i,))]
    out_shape = [jax.ShapeDtypeStruct((TX_PAD,), jnp.float32)]
  else:
    out_specs = [pspec] * NCHUNK
    out_shape = [jax.ShapeDtypeStruct((TX_PAD, CW), jnp.float32)] * NCHUNK
  return pl.pallas_call(
      functools.partial(_tx_update_body, head, nlow),
      grid=grid,
      in_specs=(
          fspecs + cfspecs
          + [pspec] * NCHUNK
          + [pl.BlockSpec((BN, 16), lambda i: (i, 0))]
          + [pspec] * NCHUNK
          + [pl.BlockSpec((5, H, H), lambda i: (0, 0, 0)),
             pl.BlockSpec((H, H), lambda i: (0, 0)),
             pl.BlockSpec((H, H), lambda i: (0, 0)),
             pl.BlockSpec((1, H), lambda i: (0, 0)),
             pl.BlockSpec((H, H), lambda i: (0, 0)),
             pl.BlockSpec((1, H), lambda i: (0, 0)),
             pl.BlockSpec((1, H), lambda i: (0, 0)),
             pl.BlockSpec((1,), lambda i: (0,))]
      ),
      out_specs=out_specs,
      out_shape=out_shape,
  )(*(list(s_fr) * 5), *([c_fr] * 5), *s_pat, c_pat, *x_tx,
    wlt_fr, wlt_pat, wrsum, bsum, wh1t, bh1, wh2, bh2)


@functools.cache
def _mesh():
  return plsc.VectorSubcoreMesh(
      core_axis_name="c", subcore_axis_name="s", num_cores=NC,
      num_subcores=NS)


def _secs(nrow, width=48):
  out = []
  off = 0
  while off < nrow:
    sz = min(width, nrow - off)
    out.append((off, sz))
    off += sz
  return out


def _zero_rows_buf(rows):
  def body(i, _):
    for b in range(4):
      rows[b, i, pl.ds(0, 16)] = jnp.zeros((16,), jnp.float32)
      rows[b, i, pl.ds(16, 16)] = jnp.zeros((16,), jnp.float32)
    return 0

  lax.fori_loop(0, 128, body, 0)


def _zero_fill(zsrc, acc, rows, s):
  per_tile = rows // NS
  base = s * per_tile
  off = 0
  while off < per_tile:
    sz = min(128, per_tile - off)
    pltpu.sync_copy(zsrc.at[pl.ds(0, sz)], acc.at[pl.ds(base + off, sz)])
    off += sz


def _agg_phase(tab, src_hbm, dst_hbm, acc, rows_acc, isrc, idst, rows,
               gsems, ssems, s):
  nrow = src_hbm.shape[1]
  _zero_rows_buf(rows)
  _zero_fill(rows.at[0], acc, rows_acc, s)
  plsc.subcore_barrier()
  for off, sz in _secs(nrow, 24):
    pltpu.sync_copy(src_hbm.at[s, pl.ds(off, sz)], isrc.at[pl.ds(0, sz)])
    pltpu.sync_copy(dst_hbm.at[s, pl.ds(off, sz)], idst.at[pl.ds(0, sz)])
    for j in range(min(2, sz)):
      pltpu.async_copy(tab.at[isrc.at[j]], rows.at[j], gsems.at[j])

    def step(j):
      b = j % 4
      bn = (j + 2) % 4

      @pl.when(j + 2 < sz)
      def _():
        @pl.when(j >= 2)
        def _():
          pltpu.make_async_copy(
              rows.at[bn], acc.at[idst.at[j - 2]], ssems.at[bn]).wait()

        pltpu.async_copy(tab.at[isrc.at[j + 2]], rows.at[bn], gsems.at[bn])

      pltpu.make_async_copy(
          tab.at[isrc.at[j]], rows.at[b], gsems.at[b]).wait()
      pltpu.async_copy(rows.at[b], acc.at[idst.at[j]], ssems.at[b], add=True)

    def body(p, _):
      step(2 * p)
      step(2 * p + 1)
      return 0

    lax.fori_loop(0, sz // 2, body, 0)
    if sz % 2:
      j = sz - 1
      pltpu.make_async_copy(
          tab.at[isrc.at[j]], rows.at[j % 4], gsems.at[j % 4]).wait()
      pltpu.async_copy(
          rows.at[j % 4], acc.at[idst.at[j]], ssems.at[j % 4], add=True)
    for k in range(max(0, sz - 4), sz):
      pltpu.make_async_copy(
          rows.at[k % 4], acc.at[idst.at[k]], ssems.at[k % 4]).wait()
  plsc.subcore_barrier()


@functools.cache
def _build_sc_layer(include_to):
  n_in = 15 if include_to else 14
  n_out = (3 if include_to else 2) * NCHUNK
  out_type = (
      ([jax.ShapeDtypeStruct((ENT_ALL, CW), jnp.float32)] * NCHUNK
       if include_to else [])
      + [jax.ShapeDtypeStruct((ENT_ALL, CW), jnp.float32)] * NCHUNK
      + [jax.ShapeDtypeStruct((TX_PAD, CW), jnp.float32)] * NCHUNK)

  def body(*refs):
    txc = refs[0:4]
    entc = refs[4:8]
    to_src, to_dst, fr_src, fr_dst, pat_src, pat_dst = refs[8:14]
    outs = refs[n_in:n_in + n_out]
    if include_to:
      s_to = outs[0:4]
      s_fr = outs[4:8]
      s_pat = outs[8:12]
    else:
      s_fr = outs[0:4]
      s_pat = outs[4:8]
    acc, isrc, idst, rows, gsems, ssems = refs[n_in + n_out:]

    c = lax.axis_index("c")
    s = lax.axis_index("s")

    for f in range(NCHUNK):
      with_f = pl.when(c == f // 2)

      if include_to:
        @with_f
        def _(f=f):
          _agg_phase(txc[f], to_src, to_dst, acc, TO_ACC, isrc, idst,
                     rows, gsems, ssems, s)
          per = ENT_ALL // NS
          pltpu.sync_copy(acc.at[pl.ds(s * per, per)],
                          s_to[f].at[pl.ds(s * per, per)])
          plsc.subcore_barrier()

      @with_f
      def _(f=f):
        _agg_phase(entc[f], fr_src, fr_dst, acc, FR_ACC, isrc, idst,
                   rows, gsems, ssems, s)
        per = ENT_PAD // NS
        for et in range(5):
          pltpu.sync_copy(
              acc.at[pl.ds(et * FR_STRIDE + s * per, per)],
              s_fr[f].at[pl.ds(et * ENT_PAD + s * per, per)])
        plsc.subcore_barrier()

      @with_f
      def _(f=f):
        _agg_phase(txc[f], pat_src, pat_dst, acc, PAT_ACC, isrc, idst,
                   rows, gsems, ssems, s)
        per = TX_PAD // NS
        pltpu.sync_copy(acc.at[pl.ds(s * per, per)],
                        s_pat[f].at[pl.ds(s * per, per)])
        plsc.subcore_barrier()

  return pl.kernel(
      body,
      out_type=out_type,
      mesh=_mesh(),
      compiler_params=pltpu.CompilerParams(use_tc_tiling_on_sc=False),
      scratch_types=[
          pltpu.VMEM_SHARED((FR_ACC, CW), jnp.float32),
          pltpu.VMEM((24, 128), jnp.int32),
          pltpu.VMEM((24, 128), jnp.int32),
          pltpu.VMEM((4, 128, CW), jnp.float32),
          pltpu.SemaphoreType.DMA((4,)),
          pltpu.SemaphoreType.DMA((4,)),
      ],
  )


def _counts_phase(dst_hbm, cacc, rows_acc, ones, zeros, idst, s):
  nrow = dst_hbm.shape[1]
  per_tile = rows_acc // NS
  base = s * per_tile
  off = 0
  while off < per_tile:
    sz = min(128, per_tile - off)
    pltpu.sync_copy(zeros.at[pl.ds(0, sz)], cacc.at[pl.ds(base + off, sz)])
    off += sz
  plsc.subcore_barrier()
  for off, sz in _secs(nrow):
    pltpu.sync_copy(dst_hbm.at[s, pl.ds(off, sz)], idst.at[pl.ds(0, sz)])

    def body(j, _):
      pltpu.sync_copy(ones, cacc.at[idst.at[j]], add=True)
      return 0

    lax.fori_loop(0, sz, body, 0)
  plsc.subcore_barrier()


@functools.cache
def _build_sc_counts():
  out_type = [
      jax.ShapeDtypeStruct((ENT_ALL, 16), jnp.float32),
      jax.ShapeDtypeStruct((ENT_ALL, 16), jnp.float32),
      jax.ShapeDtypeStruct((TX_PAD, 16), jnp.float32),
  ]

  def body(to_dst, fr_dst, pat_dst, c_to, c_fr, c_pat, cacc, idst, ones,
           zeros):
    c = lax.axis_index("c")
    s = lax.axis_index("s")

    def fill(i, _):
      ones[i, :] = jnp.ones((16,), jnp.float32)
      zeros[i, :] = jnp.zeros((16,), jnp.float32)
      return 0

    lax.fori_loop(0, 128, fill, 0)
    plsc.subcore_barrier()

    @pl.when(c == 0)
    def _():
      _counts_phase(to_dst, cacc, TO_ACC, ones, zeros, idst, s)
      per = ENT_ALL // NS
      pltpu.sync_copy(cacc.at[pl.ds(s * per, per)],
                      c_to.at[pl.ds(s * per, per)])
      plsc.subcore_barrier()

    @pl.when(c == 1)
    def _():
      _counts_phase(fr_dst, cacc, FR_ACC, ones, zeros, idst, s)
      per = ENT_PAD // NS
      for et in range(5):
        pltpu.sync_copy(
            cacc.at[pl.ds(et * FR_STRIDE + s * per, per)],
            c_fr.at[pl.ds(et * ENT_PAD + s * per, per)])
      plsc.subcore_barrier()
      _counts_phase(pat_dst, cacc, PAT_ACC, ones, zeros, idst, s)
      per = TX_PAD // NS
      pltpu.sync_copy(cacc.at[pl.ds(s * per, per)],
                      c_pat.at[pl.ds(s * per, per)])
      plsc.subcore_barrier()

  return pl.kernel(
      body,
      out_type=out_type,
      mesh=_mesh(),
      compiler_params=pltpu.CompilerParams(use_tc_tiling_on_sc=False),
      scratch_types=[
          pltpu.VMEM_SHARED((FR_ACC, 16), jnp.float32),
          pltpu.VMEM((48, 128), jnp.int32),
          pltpu.VMEM((128, 16), jnp.float32),
          pltpu.VMEM((128, 16), jnp.float32),
      ],
  )


def _pad_rows(x, n):
  return jnp.pad(x, ((0, n - x.shape[0]),) + ((0, 0),) * (x.ndim - 1))


def _edges(eis, offs_dst, dump, total, offs_src=None):
  srcs, dsts = [], []
  for k, ei in enumerate(eis):
    s = ei[0].astype(jnp.int32)
    if offs_src is not None:
      s = s + offs_src[k]
    srcs.append(s)
    dsts.append(ei[1].astype(jnp.int32) + offs_dst[k])
  ns = jnp.concatenate(srcs) if len(srcs) > 1 else srcs[0]
  nd = jnp.concatenate(dsts) if len(dsts) > 1 else dsts[0]
  pad = total - ns.shape[0]
  ns = jnp.concatenate([ns, jnp.zeros((pad,), jnp.int32)])
  nd = jnp.concatenate([nd, jnp.full((pad,), dump, jnp.int32)])
  return ns.reshape(NS, -1, 128), nd.reshape(NS, -1, 128)


def kernel(tx_x, card_x, addr_x, email_x, device_x, product_x, ei_to_card,
           ei_from_card, ei_to_addr, ei_from_addr, ei_to_email, ei_from_email,
           ei_to_device, ei_from_device, ei_to_product, ei_from_product,
           ei_tx_pattern, params):
  ents = [card_x, addr_x, email_x, device_x, product_x]
  ei_to = [ei_to_card, ei_to_addr, ei_to_email, ei_to_device, ei_to_product]
  ei_fr = [ei_from_card, ei_from_addr, ei_from_email, ei_from_device,
           ei_from_product]

  tx_pad = _pad_rows(tx_x, TX_PAD)
  ent_all = jnp.concatenate([_pad_rows(e, ENT_PAD) for e in ents])

  to_src, to_dst = _edges(
      ei_to, [et * ENT_PAD for et in range(5)], TO_DUMP, TO_E)
  fr_src, fr_dst = _edges(
      ei_fr, [et * FR_STRIDE for et in range(5)], FR_DUMP, FR_E,
      offs_src=[et * ENT_PAD for et in range(5)])
  pat_src, pat_dst = _edges([ei_tx_pattern], [0], PAT_DUMP, PAT_E)

  p = params
  w1t, b1 = p["tx_proj"][0].T, p["tx_proj"][1][None, :]
  w2t, b2 = p["tx_proj"][2].T, p["tx_proj"][3][None, :]
  wet, be = p["entity_proj"][0].T, p["entity_proj"][1][None, :]
  wh1t, bh1 = p["head"][0].T, p["head"][1][None, :]
  wh2, bh2 = p["head"][2], p["head"][3]

  ets = ["card", "addr", "email", "device", "product"]
  layer_w = []
  for layer in p["layers"]:
    wlt_to = jnp.stack([layer["to_" + et][0].T for et in ets])
    bl_to = jnp.stack([layer["to_" + et][1] for et in ets])[:, None, :]
    wrt_to = jnp.stack([layer["to_" + et][2].T for et in ets])
    wlt_fr = jnp.stack([layer["from_" + et][0].T / 6.0 for et in ets])
    wlt_pat = layer["tx_pattern"][0].T / 6.0
    wr_sum = sum(layer["from_" + et][2] for et in ets) + \
        layer["tx_pattern"][2]
    wrsum_t = wr_sum.T / 6.0
    bsum = (sum(layer["from_" + et][1] for et in ets)
            + layer["tx_pattern"][1])[None, :] / 6.0
    layer_w.append((wlt_to, bl_to, wrt_to, wlt_fr, wlt_pat, wrsum_t, bsum))

  x_tx = _tx_proj(tx_pad, w1t, b1, w2t, b2)
  x_ent = _ent_proj(ent_all, wet, be)

  c_to, c_fr, c_pat = _build_sc_counts()(to_dst, fr_dst, pat_dst)

  dummy = jnp.zeros((1, H), jnp.float32)
  dummy1 = jnp.zeros((1,), jnp.float32)

  wlt_to, bl_to, wrt_to, wlt_fr, wlt_pat, wrsum_t, bsum = layer_w[0]
  outs = _build_sc_layer(True)(*x_tx, *x_ent, to_src, to_dst, fr_src,
                               fr_dst, pat_src, pat_dst, c_pat)
  s_to, s_fr, s_pat = outs[0:4], outs[4:8], outs[8:12]
  x_ent1 = _ent_update(s_to, c_to, x_ent, wlt_to, bl_to, wrt_to)
  x_tx1 = _tx_update(False, s_fr, c_fr, s_pat, c_pat, x_tx, wlt_fr, wlt_pat,
                     wrsum_t, bsum, dummy, dummy, dummy, dummy1)

  _, _, _, wlt_fr, wlt_pat, wrsum_t, bsum = layer_w[1]
  outs = _build_sc_layer(False)(*x_tx1, *x_ent1, to_src, to_dst, fr_src, fr_dst,
                    pat_src, pat_dst)
  s_fr, s_pat = outs[0:4], outs[4:8]
  (out,) = _tx_update(True, s_fr, c_fr, s_pat, c_pat, x_tx1, wlt_fr, wlt_pat,
                      wrsum_t, bsum, wh1t, bh1, wh2, bh2)
  return out[:N_TX]


# speedup vs baseline: 3.2682x; 1.0629x over previous
import functools
import jax
import jax.numpy as jnp
from jax import lax
from jax.experimental import pallas as pl
from jax.experimental.pallas import tpu as pltpu
from jax.experimental.pallas import tpu_sc as plsc

N_TX = 50000
N_ENT = 10000
H = 128
NCHUNK = 4
CW = 32
NS = 16
NC = 2

TX_PAD = 50176
ENT_PAD = 10240
ENT_ALL = 5 * ENT_PAD

TO_ACC = 51328
FR_STRIDE = 10248
FR_ACC = 51328
PAT_ACC = 50304

TO_DUMP = TO_ACC - 1
FR_DUMP = FR_ACC - 1
PAT_DUMP = PAT_ACC - 1

TO_E = 251904
FR_E = 251904
PAT_E = 100352

BN = 1024


def _elu(x):
  return jnp.where(x > 0, x, jnp.exp(x) - 1.0)


def _inv_counts(cblk):
  c = cblk[:, 0:1]
  return 1.0 / jnp.maximum(c, 1.0)


def _chunk_writes(y, out_refs):
  for f in range(NCHUNK):
    out_refs[f][...] = y[:, f * CW:(f + 1) * CW]


def _concat_chunks(refs):
  return jnp.concatenate([refs[f][...] for f in range(NCHUNK)], axis=1)


def _tx_proj_body(x_ref, w1_ref, b1_ref, w2_ref, b2_ref, *out_refs):
  h = jnp.maximum(
      jnp.dot(x_ref[...], w1_ref[...], preferred_element_type=jnp.float32)
      + b1_ref[...], 0.0)
  y = _elu(
      jnp.dot(h, w2_ref[...], preferred_element_type=jnp.float32)
      + b2_ref[...])
  _chunk_writes(y, out_refs)


def _tx_proj(tx_pad, w1t, b1, w2t, b2):
  grid = TX_PAD // BN
  return pl.pallas_call(
      _tx_proj_body,
      grid=(grid,),
      in_specs=[
          pl.BlockSpec((BN, H), lambda i: (i, 0)),
          pl.BlockSpec((H, H), lambda i: (0, 0)),
          pl.BlockSpec((1, H), lambda i: (0, 0)),
          pl.BlockSpec((H, H), lambda i: (0, 0)),
          pl.BlockSpec((1, H), lambda i: (0, 0)),
      ],
      out_specs=[pl.BlockSpec((BN, CW), lambda i: (i, 0))] * NCHUNK,
      out_shape=[jax.ShapeDtypeStruct((TX_PAD, CW), jnp.float32)] * NCHUNK,
  )(tx_pad, w1t, b1, w2t, b2)


def _ent_proj_body(x_ref, we_ref, be_ref, *out_refs):
  y = jnp.maximum(
      jnp.dot(x_ref[...], we_ref[...], preferred_element_type=jnp.float32)
      + be_ref[...], 0.0)
  _chunk_writes(y, out_refs)


def _ent_proj(ent_all, wet, be):
  bne = 2048
  grid = ENT_ALL // bne
  return pl.pallas_call(
      _ent_proj_body,
      grid=(grid,),
      in_specs=[
          pl.BlockSpec((bne, 2), lambda i: (i, 0)),
          pl.BlockSpec((2, H), lambda i: (0, 0)),
          pl.BlockSpec((1, H), lambda i: (0, 0)),
      ],
      out_specs=[pl.BlockSpec((bne, CW), lambda i: (i, 0))] * NCHUNK,
      out_shape=[jax.ShapeDtypeStruct((ENT_ALL, CW), jnp.float32)] * NCHUNK,
  )(ent_all, wet, be)


def _ent_update_body(s0, s1, s2, s3, cto, x0, x1, x2, x3, wlt, bl, wrt,
                     *out_refs):
  agg = _concat_chunks((s0, s1, s2, s3)) * _inv_counts(cto[...])
  x = _concat_chunks((x0, x1, x2, x3))
  y = _elu(
      jnp.dot(agg, wlt[0], preferred_element_type=jnp.float32)
      + bl[0]
      + jnp.dot(x, wrt[0], preferred_element_type=jnp.float32))
  _chunk_writes(y, out_refs)


def _ent_update(s_to, c_to, x_ent, wlt_to, bl_to, wrt_to):
  grid = ENT_ALL // BN
  blocks_per_et = ENT_PAD // BN
  sspec = pl.BlockSpec((BN, CW), lambda i: (i, 0))
  return pl.pallas_call(
      _ent_update_body,
      grid=(grid,),
      in_specs=(
          [sspec] * NCHUNK
          + [pl.BlockSpec((BN, 16), lambda i: (i, 0))]
          + [sspec] * NCHUNK
          + [pl.BlockSpec((1, H, H), lambda i: (i // blocks_per_et, 0, 0)),
             pl.BlockSpec((1, 1, H), lambda i: (i // blocks_per_et, 0, 0)),
             pl.BlockSpec((1, H, H), lambda i: (i // blocks_per_et, 0, 0))]
      ),
      out_specs=[sspec] * NCHUNK,
      out_shape=[jax.ShapeDtypeStruct((ENT_ALL, CW), jnp.float32)] * NCHUNK,
  )(*s_to, c_to, *x_ent, wlt_to, bl_to, wrt_to)


def _tx_update_body(head, nlow, *refs):
  frs = refs[0:20]
  cfrs = refs[20:25]
  ps = refs[25:29]
  cpat = refs[29]
  xs = refs[30:34]
  (wlt_fr, wlt_pat, wrsum, bsum, wh1t, bh1, wh2, bh2) = refs[34:42]
  out_refs = refs[42:]
  i = pl.program_id(0)

  x = _concat_chunks(xs)
  sp = _concat_chunks(ps) * _inv_counts(cpat[...])
  acc = (jnp.dot(sp, wlt_pat[...], preferred_element_type=jnp.float32)
         + jnp.dot(x, wrsum[...], preferred_element_type=jnp.float32)
         + bsum[...])

  @pl.when(i < nlow)
  def _():
    z = acc
    for et in range(5):
      sf = _concat_chunks(frs[4 * et:4 * et + 4]) * _inv_counts(
          cfrs[et][...])
      z = z + jnp.dot(sf, wlt_fr[et],
                      preferred_element_type=jnp.float32)
    _tx_update_out(head, z, out_refs, wh1t, bh1, wh2, bh2)

  @pl.when(i >= nlow)
  def _():
    _tx_update_out(head, acc, out_refs, wh1t, bh1, wh2, bh2)


def _tx_update_out(head, z, out_refs, wh1t, bh1, wh2, bh2):
  y = _elu(z)
  if head:
    h = jnp.maximum(
        jnp.dot(y, wh1t[...], preferred_element_type=jnp.float32)
        + bh1[...], 0.0)
    o = jnp.sum(h * wh2[...], axis=1) + bh2[0]
    out_refs[0][...] = o
  else:
    _chunk_writes(y, out_refs)


def _tx_update(head, s_fr, c_fr, s_pat, c_pat, x_tx, wlt_fr, wlt_pat, wrsum,
               bsum, wh1t, bh1, wh2, bh2):
  grid = (TX_PAD // BN,)
  nlow = ENT_PAD // BN
  fspecs, cfspecs = [], []
  for et in range(5):
    fspecs += [pl.BlockSpec(
        (BN, CW),
        lambda i, et=et: (et * nlow + jnp.minimum(i, nlow - 1), 0))] * NCHUNK
    cfspecs += [pl.BlockSpec(
        (BN, 16),
        lambda i, et=et: (et * nlow + jnp.minimum(i, nlow - 1), 0))]
  pspec = pl.BlockSpec((BN, CW), lambda i: (i, 0))
  if head:
    out_specs = [pl.BlockSpec((BN,), lambda i: (i,))]
    out_shape = [jax.ShapeDtypeStruct((TX_PAD,), jnp.float32)]
  else:
    out_specs = [pspec] * NCHUNK
    out_shape = [jax.ShapeDtypeStruct((TX_PAD, CW), jnp.float32)] * NCHUNK
  return pl.pallas_call(
      functools.partial(_tx_update_body, head, nlow),
      grid=grid,
      in_specs=(
          fspecs + cfspecs
          + [pspec] * NCHUNK
          + [pl.BlockSpec((BN, 16), lambda i: (i, 0))]
          + [pspec] * NCHUNK
          + [pl.BlockSpec((5, H, H), lambda i: (0, 0, 0)),
             pl.BlockSpec((H, H), lambda i: (0, 0)),
             pl.BlockSpec((H, H), lambda i: (0, 0)),
             pl.BlockSpec((1, H), lambda i: (0, 0)),
             pl.BlockSpec((H, H), lambda i: (0, 0)),
             pl.BlockSpec((1, H), lambda i: (0, 0)),
             pl.BlockSpec((1, H), lambda i: (0, 0)),
             pl.BlockSpec((1,), lambda i: (0,))]
      ),
      out_specs=out_specs,
      out_shape=out_shape,
  )(*(list(s_fr) * 5), *([c_fr] * 5), *s_pat, c_pat, *x_tx,
    wlt_fr, wlt_pat, wrsum, bsum, wh1t, bh1, wh2, bh2)


@functools.cache
def _mesh():
  return plsc.VectorSubcoreMesh(
      core_axis_name="c", subcore_axis_name="s", num_cores=NC,
      num_subcores=NS)


def _secs(nrow, width=48):
  out = []
  off = 0
  while off < nrow:
    sz = min(width, nrow - off)
    out.append((off, sz))
    off += sz
  return out


def _zero_rows_buf(rows):
  def body(i, _):
    for b in range(4):
      rows[b, i, pl.ds(0, 16)] = jnp.zeros((16,), jnp.float32)
      rows[b, i, pl.ds(16, 16)] = jnp.zeros((16,), jnp.float32)
    return 0

  lax.fori_loop(0, 128, body, 0)


def _zero_fill(zsrc, acc, rows, s):
  per_tile = rows // NS
  base = s * per_tile
  off = 0
  while off < per_tile:
    sz = min(128, per_tile - off)
    pltpu.sync_copy(zsrc.at[pl.ds(0, sz)], acc.at[pl.ds(base + off, sz)])
    off += sz


def _agg_phase(tab, src_hbm, dst_hbm, acc, rows_acc, isrc, idst, rows,
               gsems, ssems, s):
  nrow = src_hbm.shape[1]
  _zero_rows_buf(rows)
  _zero_fill(rows.at[0], acc, rows_acc, s)
  plsc.subcore_barrier()
  for off, sz in _secs(nrow, 24):
    pltpu.sync_copy(src_hbm.at[s, pl.ds(off, sz)], isrc.at[pl.ds(0, sz)])
    pltpu.sync_copy(dst_hbm.at[s, pl.ds(off, sz)], idst.at[pl.ds(0, sz)])
    for j in range(min(2, sz)):
      pltpu.async_copy(tab.at[isrc.at[j]], rows.at[j], gsems.at[j])

    def step(j):
      b = j % 4
      bn = (j + 2) % 4

      @pl.when(j + 2 < sz)
      def _():
        @pl.when(j >= 2)
        def _():
          pltpu.make_async_copy(
              rows.at[bn], acc.at[idst.at[j - 2]], ssems.at[bn]).wait()

        pltpu.async_copy(tab.at[isrc.at[j + 2]], rows.at[bn], gsems.at[bn])

      pltpu.make_async_copy(
          tab.at[isrc.at[j]], rows.at[b], gsems.at[b]).wait()
      pltpu.async_copy(rows.at[b], acc.at[idst.at[j]], ssems.at[b], add=True)

    def body(p, _):
      step(2 * p)
      step(2 * p + 1)
      return 0

    lax.fori_loop(0, sz // 2, body, 0)
    if sz % 2:
      j = sz - 1
      pltpu.make_async_copy(
          tab.at[isrc.at[j]], rows.at[j % 4], gsems.at[j % 4]).wait()
      pltpu.async_copy(
          rows.at[j % 4], acc.at[idst.at[j]], ssems.at[j % 4], add=True)
    for k in range(max(0, sz - 4), sz):
      pltpu.make_async_copy(
          rows.at[k % 4], acc.at[idst.at[k]], ssems.at[k % 4]).wait()
  plsc.subcore_barrier()


@functools.cache
def _build_sc_layer(include_to):
  n_in = 15 if include_to else 14
  n_out = (3 if include_to else 2) * NCHUNK
  out_type = (
      ([jax.ShapeDtypeStruct((ENT_ALL, CW), jnp.float32)] * NCHUNK
       if include_to else [])
      + [jax.ShapeDtypeStruct((ENT_ALL, CW), jnp.float32)] * NCHUNK
      + [jax.ShapeDtypeStruct((TX_PAD, CW), jnp.float32)] * NCHUNK)

  def body(*refs):
    txc = refs[0:4]
    entc = refs[4:8]
    to_src, to_dst, fr_src, fr_dst, pat_src, pat_dst = refs[8:14]
    outs = refs[n_in:n_in + n_out]
    if include_to:
      s_to = outs[0:4]
      s_fr = outs[4:8]
      s_pat = outs[8:12]
    else:
      s_fr = outs[0:4]
      s_pat = outs[4:8]
    acc, isrc, idst, rows, gsems, ssems = refs[n_in + n_out:]

    c = lax.axis_index("c")
    s = lax.axis_index("s")

    for f in range(NCHUNK):
      with_f = pl.when(c == f // 2)

      if include_to:
        @with_f
        def _(f=f):
          _agg_phase(txc[f], to_src, to_dst, acc, TO_ACC, isrc, idst,
                     rows, gsems, ssems, s)
          per = ENT_ALL // NS
          pltpu.sync_copy(acc.at[pl.ds(s * per, per)],
                          s_to[f].at[pl.ds(s * per, per)])
          plsc.subcore_barrier()

      @with_f
      def _(f=f):
        _agg_phase(entc[f], fr_src, fr_dst, acc, FR_ACC, isrc, idst,
                   rows, gsems, ssems, s)
        per = ENT_PAD // NS
        for et in range(5):
          pltpu.sync_copy(
              acc.at[pl.ds(et * FR_STRIDE + s * per, per)],
              s_fr[f].at[pl.ds(et * ENT_PAD + s * per, per)])
        plsc.subcore_barrier()

      @with_f
      def _(f=f):
        _agg_phase(txc[f], pat_src, pat_dst, acc, PAT_ACC, isrc, idst,
                   rows, gsems, ssems, s)
        per = TX_PAD // NS
        pltpu.sync_copy(acc.at[pl.ds(s * per, per)],
                        s_pat[f].at[pl.ds(s * per, per)])
        plsc.subcore_barrier()

  return pl.kernel(
      body,
      out_type=out_type,
      mesh=_mesh(),
      compiler_params=pltpu.CompilerParams(use_tc_tiling_on_sc=False),
      scratch_types=[
          pltpu.VMEM_SHARED((FR_ACC, CW), jnp.float32),
          pltpu.VMEM((24, 128), jnp.int32),
          pltpu.VMEM((24, 128), jnp.int32),
          pltpu.VMEM((4, 128, CW), jnp.float32),
          pltpu.SemaphoreType.DMA((4,)),
          pltpu.SemaphoreType.DMA((4,)),
      ],
  )


def _counts_phase(dst_hbm, cacc, rows_acc, ones, zeros, idst, s):
  nrow = dst_hbm.shape[1]
  per_tile = rows_acc // NS
  base = s * per_tile
  off = 0
  while off < per_tile:
    sz = min(128, per_tile - off)
    pltpu.sync_copy(zeros.at[pl.ds(0, sz)], cacc.at[pl.ds(base + off, sz)])
    off += sz
  plsc.subcore_barrier()
  for off, sz in _secs(nrow):
    pltpu.sync_copy(dst_hbm.at[s, pl.ds(off, sz)], idst.at[pl.ds(0, sz)])

    def body(j, _):
      pltpu.sync_copy(ones, cacc.at[idst.at[j]], add=True)
      return 0

    lax.fori_loop(0, sz, body, 0)
  plsc.subcore_barrier()


@functools.cache
def _build_sc_counts():
  out_type = [
      jax.ShapeDtypeStruct((ENT_ALL, 16), jnp.float32),
      jax.ShapeDtypeStruct((ENT_ALL, 16), jnp.float32),
      jax.ShapeDtypeStruct((TX_PAD, 16), jnp.float32),
  ]

  def body(to_dst, fr_dst, pat_dst, c_to, c_fr, c_pat, cacc, idst, ones,
           zeros):
    c = lax.axis_index("c")
    s = lax.axis_index("s")

    def fill(i, _):
      ones[i, :] = jnp.ones((16,), jnp.float32)
      zeros[i, :] = jnp.zeros((16,), jnp.float32)
      return 0

    lax.fori_loop(0, 128, fill, 0)
    plsc.subcore_barrier()

    @pl.when(c == 0)
    def _():
      _counts_phase(to_dst, cacc, TO_ACC, ones, zeros, idst, s)
      per = ENT_ALL // NS
      pltpu.sync_copy(cacc.at[pl.ds(s * per, per)],
                      c_to.at[pl.ds(s * per, per)])
      plsc.subcore_barrier()

    @pl.when(c == 1)
    def _():
      _counts_phase(fr_dst, cacc, FR_ACC, ones, zeros, idst, s)
      per = ENT_PAD // NS
      for et in range(5):
        pltpu.sync_copy(
            cacc.at[pl.ds(et * FR_STRIDE + s * per, per)],
            c_fr.at[pl.ds(et * ENT_PAD + s * per, per)])
      plsc.subcore_barrier()
      _counts_phase(pat_dst, cacc, PAT_ACC, ones, zeros, idst, s)
      per = TX_PAD // NS
      pltpu.sync_copy(cacc.at[pl.ds(s * per, per)],
                      c_pat.at[pl.ds(s * per, per)])
      plsc.subcore_barrier()

  return pl.kernel(
      body,
      out_type=out_type,
      mesh=_mesh(),
      compiler_params=pltpu.CompilerParams(use_tc_tiling_on_sc=False),
      scratch_types=[
          pltpu.VMEM_SHARED((FR_ACC, 16), jnp.float32),
          pltpu.VMEM((48, 128), jnp.int32),
          pltpu.VMEM((128, 16), jnp.float32),
          pltpu.VMEM((128, 16), jnp.float32),
      ],
  )


def _pad_rows(x, n):
  return jnp.pad(x, ((0, n - x.shape[0]),) + ((0, 0),) * (x.ndim - 1))


def _edges(eis, offs_dst, dump, total, offs_src=None):
  srcs, dsts = [], []
  for k, ei in enumerate(eis):
    s = ei[0].astype(jnp.int32)
    if offs_src is not None:
      s = s + offs_src[k]
    srcs.append(s)
    dsts.append(ei[1].astype(jnp.int32) + offs_dst[k])
  ns = jnp.concatenate(srcs) if len(srcs) > 1 else srcs[0]
  nd = jnp.concatenate(dsts) if len(dsts) > 1 else dsts[0]
  pad = total - ns.shape[0]
  ns = jnp.concatenate([ns, jnp.zeros((pad,), jnp.int32)])
  nd = jnp.concatenate([nd, jnp.full((pad,), dump, jnp.int32)])
  return ns.reshape(NS, -1, 128), nd.reshape(NS, -1, 128)


def kernel(tx_x, card_x, addr_x, email_x, device_x, product_x, ei_to_card,
           ei_from_card, ei_to_addr, ei_from_addr, ei_to_email, ei_from_email,
           ei_to_device, ei_from_device, ei_to_product, ei_from_product,
           ei_tx_pattern, params):
  ents = [card_x, addr_x, email_x, device_x, product_x]
  ei_to = [ei_to_card, ei_to_addr, ei_to_email, ei_to_device, ei_to_product]
  ei_fr = [ei_from_card, ei_from_addr, ei_from_email, ei_from_device,
           ei_from_product]

  tx_pad = _pad_rows(tx_x, TX_PAD)
  ent_all = jnp.concatenate([_pad_rows(e, ENT_PAD) for e in ents])

  to_src, to_dst = _edges(
      ei_to, [et * ENT_PAD for et in range(5)], TO_DUMP, TO_E)
  fr_src, fr_dst = _edges(
      ei_fr, [et * FR_STRIDE for et in range(5)], FR_DUMP, FR_E,
      offs_src=[et * ENT_PAD for et in range(5)])
  pat_src, pat_dst = _edges([ei_tx_pattern], [0], PAT_DUMP, PAT_E)

  p = params
  w1t, b1 = p["tx_proj"][0].T, p["tx_proj"][1][None, :]
  w2t, b2 = p["tx_proj"][2].T, p["tx_proj"][3][None, :]
  wet, be = p["entity_proj"][0].T, p["entity_proj"][1][None, :]
  wh1t, bh1 = p["head"][0].T, p["head"][1][None, :]
  wh2, bh2 = p["head"][2], p["head"][3]

  ets = ["card", "addr", "email", "device", "product"]
  layer_w = []
  for layer in p["layers"]:
    wlt_to = jnp.stack([layer["to_" + et][0].T for et in ets])
    bl_to = jnp.stack([layer["to_" + et][1] for et in ets])[:, None, :]
    wrt_to = jnp.stack([layer["to_" + et][2].T for et in ets])
    wlt_fr = jnp.stack([layer["from_" + et][0].T / 6.0 for et in ets])
    wlt_pat = layer["tx_pattern"][0].T / 6.0
    wr_sum = sum(layer["from_" + et][2] for et in ets) + \
        layer["tx_pattern"][2]
    wrsum_t = wr_sum.T / 6.0
    bsum = (sum(layer["from_" + et][1] for et in ets)
            + layer["tx_pattern"][1])[None, :] / 6.0
    layer_w.append((wlt_to, bl_to, wrt_to, wlt_fr, wlt_pat, wrsum_t, bsum))

  x_tx = _tx_proj(tx_pad, w1t, b1, w2t, b2)
  x_ent = _ent_proj(ent_all, wet, be)

  c_to, c_fr, c_pat = _build_sc_counts()(to_dst, fr_dst, pat_dst)

  dummy = jnp.zeros((1, H), jnp.float32)
  dummy1 = jnp.zeros((1,), jnp.float32)

  wlt_to, bl_to, wrt_to, wlt_fr, wlt_pat, wrsum_t, bsum = layer_w[0]
  outs = _build_sc_layer(True)(*x_tx, *x_ent, to_src, to_dst, fr_src,
                               fr_dst, pat_src, pat_dst, c_pat)
  s_to, s_fr, s_pat = outs[0:4], outs[4:8], outs[8:12]
  x_ent1 = _ent_update(s_to, c_to, x_ent, wlt_to, bl_to, wrt_to)
  x_tx1 = _tx_update(False, s_fr, c_fr, s_pat, c_pat, x_tx, wlt_fr, wlt_pat,
                     wrsum_t, bsum, dummy, dummy, dummy, dummy1)

  _, _, _, wlt_fr, wlt_pat, wrsum_t, bsum = layer_w[1]
  outs = _build_sc_layer(False)(*x_tx1, *x_ent1, to_src, to_dst, fr_src, fr_dst,
                    pat_src, pat_dst)
  s_fr, s_pat = outs[0:4], outs[4:8]
  (out,) = _tx_update(True, s_fr, c_fr, s_pat, c_pat, x_tx1, wlt_fr, wlt_pat,
                      wrsum_t, bsum, wh1t, bh1, wh2, bh2)
  return out[:N_TX]
